# baseline (speedup 1.0000x reference)
import functools

import jax
import jax.numpy as jnp
from jax import lax
from jax.experimental import pallas as pl
from jax.experimental.pallas import tpu as pltpu
from jax.experimental.pallas import tpu_sc as plsc

_B = 64
_H = 2048
_V = 32000
_K = 8
_VB = 1280
_NB = _V // _VB

_NEG_INF = float("-inf")
_IMAX = 2**31 - 1


def _extract_topk(vals, idxs, k):
    outs_v, outs_i = [], []
    for _ in range(k):
        mv = jnp.max(vals, axis=1, keepdims=True)
        mi = jnp.min(jnp.where(vals == mv, idxs, _IMAX), axis=1, keepdims=True)
        outs_v.append(mv)
        outs_i.append(mi)
        vals = jnp.where(idxs == mi, _NEG_INF, vals)
    return jnp.concatenate(outs_v, axis=1), jnp.concatenate(outs_i, axis=1)


def _tc_body(hs_ref, w_ref, topi_out, scores_out, m_ref, s_ref, tv_ref, ti_ref):
    j = pl.program_id(0)

    @pl.when(j == 0)
    def _():
        m_ref[...] = jnp.full((_B, 128), -jnp.inf, jnp.float32)
        s_ref[...] = jnp.zeros((_B, 128), jnp.float32)
        tv_ref[...] = jnp.full((_B, 128), -jnp.inf, jnp.float32)
        ti_ref[...] = jnp.zeros((_B, 128), jnp.int32)

    block = lax.dot_general(
        hs_ref[...], w_ref[...],
        (((1,), (1,)), ((), ())),
        preferred_element_type=jnp.float32,
    )

    m = m_ref[:, 0:1]
    bm = jnp.max(block, axis=1, keepdims=True)
    new_m = jnp.maximum(m, bm)
    s = s_ref[:, 0:1] * jnp.exp(m - new_m) + jnp.sum(
        jnp.exp(block - new_m), axis=1, keepdims=True)
    m_ref[:, 0:1] = new_m
    s_ref[:, 0:1] = s

    gidx = j * _VB + lax.broadcasted_iota(jnp.int32, (_B, _VB), 1)
    bv, bi = _extract_topk(block, gidx, _K)
    cv = jnp.concatenate([tv_ref[:, :_K], bv], axis=1)
    ci = jnp.concatenate([ti_ref[:, :_K], bi], axis=1)
    nv, ni = _extract_topk(cv, ci, _K)
    tv_ref[:, :_K] = nv
    ti_ref[:, :_K] = ni

    @pl.when(j == _NB - 1)
    def _():
        lse = new_m + jnp.log(s)
        scores_out[...] = nv - lse
        topi_out[...] = ni


def _tc_topk(hidden_states, w_lm):
    return pl.pallas_call(
        _tc_body,
        grid=(_NB,),
        in_specs=[
            pl.BlockSpec((_B, _H), lambda j: (0, 0)),
            pl.BlockSpec((_VB, _H), lambda j: (j, 0)),
        ],
        out_specs=[
            pl.BlockSpec((_B, _K), lambda j: (0, 0)),
            pl.BlockSpec((_B, _K), lambda j: (0, 0)),
        ],
        out_shape=[
            jax.ShapeDtypeStruct((_B, _K), jnp.int32),
            jax.ShapeDtypeStruct((_B, _K), jnp.float32),
        ],
        scratch_shapes=[
            pltpu.VMEM((_B, 128), jnp.float32),
            pltpu.VMEM((_B, 128), jnp.float32),
            pltpu.VMEM((_B, 128), jnp.float32),
            pltpu.VMEM((_B, 128), jnp.int32),
        ],
        compiler_params=pltpu.CompilerParams(
            dimension_semantics=("arbitrary",)),
    )(hidden_states, w_lm)


_N_IDX = _B * _K


def _sc_remap_body(d2t_hbm, idx_hbm, out_hbm, idx_v, val_v, out_v, sem):
    wid = lax.axis_index("s") * 2 + lax.axis_index("c")
    base = wid * 16
    pltpu.sync_copy(idx_hbm.at[pl.ds(base, 16)], idx_v)
    pltpu.async_copy(d2t_hbm.at[idx_v], val_v, sem).wait()
    out_v[...] = idx_v[...] + val_v[...]
    pltpu.sync_copy(out_v, out_hbm.at[pl.ds(base, 16)])


@functools.cache
def _sc_remap():
    mesh = plsc.VectorSubcoreMesh(core_axis_name="c", subcore_axis_name="s")
    return pl.kernel(
        _sc_remap_body,
        mesh=mesh,
        out_type=jax.ShapeDtypeStruct((_N_IDX,), jnp.int32),
        scratch_types=[
            pltpu.VMEM((16,), jnp.int32),
            pltpu.VMEM((16,), jnp.int32),
            pltpu.VMEM((16,), jnp.int32),
            pltpu.SemaphoreType.DMA,
        ],
    )


def kernel(hidden_states, d2t, W_lm):
    topi, scores = _tc_topk(hidden_states, W_lm)
    mapped = _sc_remap()(d2t, topi.reshape(-1)).reshape(_B, _K)
    return mapped, scores


# speedup vs baseline: 1.4480x; 1.4480x over previous
import functools

import jax
import jax.numpy as jnp
from jax import lax
from jax.experimental import pallas as pl
from jax.experimental.pallas import tpu as pltpu
from jax.experimental.pallas import tpu_sc as plsc

_B = 64
_H = 2048
_V = 32000
_K = 8
_VB = 3200
_NB = _V // _VB

_NEG_INF = float("-inf")
_IMAX = 2**31 - 1


def _extract_topk(vals, idxs, k):
    outs_v, outs_i = [], []
    for _ in range(k):
        mv = jnp.max(vals, axis=1, keepdims=True)
        mi = jnp.min(jnp.where(vals == mv, idxs, _IMAX), axis=1, keepdims=True)
        outs_v.append(mv)
        outs_i.append(mi)
        vals = jnp.where(idxs == mi, _NEG_INF, vals)
    return jnp.concatenate(outs_v, axis=1), jnp.concatenate(outs_i, axis=1)


def _tc_body(hs_ref, w_ref, topi_out, scores_out, m_ref, s_ref, tv_ref, ti_ref):
    j = pl.program_id(0)

    @pl.when(j == 0)
    def _():
        m_ref[...] = jnp.full((_B, 128), -jnp.inf, jnp.float32)
        s_ref[...] = jnp.zeros((_B, 128), jnp.float32)
        tv_ref[...] = jnp.full((_B, 128), -jnp.inf, jnp.float32)
        ti_ref[...] = jnp.zeros((_B, 128), jnp.int32)

    block = lax.dot_general(
        hs_ref[...], w_ref[...],
        (((1,), (1,)), ((), ())),
        preferred_element_type=jnp.float32,
    )

    m = m_ref[:, 0:1]
    bm = jnp.max(block, axis=1, keepdims=True)
    new_m = jnp.maximum(m, bm)
    s = s_ref[:, 0:1] * jnp.exp(m - new_m) + jnp.sum(
        jnp.exp(block - new_m), axis=1, keepdims=True)
    m_ref[:, 0:1] = new_m
    s_ref[:, 0:1] = s

    gidx = j * _VB + lax.broadcasted_iota(jnp.int32, (_B, _VB), 1)
    bv, bi = _extract_topk(block, gidx, _K)
    cv = jnp.concatenate([tv_ref[:, :_K], bv], axis=1)
    ci = jnp.concatenate([ti_ref[:, :_K], bi], axis=1)
    nv, ni = _extract_topk(cv, ci, _K)
    tv_ref[:, :_K] = nv
    ti_ref[:, :_K] = ni

    @pl.when(j == _NB - 1)
    def _():
        lse = new_m + jnp.log(s)
        scores_out[...] = nv - lse
        topi_out[...] = ni


def _tc_topk(hidden_states, w_lm):
    return pl.pallas_call(
        _tc_body,
        grid=(_NB,),
        in_specs=[
            pl.BlockSpec((_B, _H), lambda j: (0, 0)),
            pl.BlockSpec((_VB, _H), lambda j: (j, 0)),
        ],
        out_specs=[
            pl.BlockSpec((_B, _K), lambda j: (0, 0)),
            pl.BlockSpec((_B, _K), lambda j: (0, 0)),
        ],
        out_shape=[
            jax.ShapeDtypeStruct((_B, _K), jnp.int32),
            jax.ShapeDtypeStruct((_B, _K), jnp.float32),
        ],
        scratch_shapes=[
            pltpu.VMEM((_B, 128), jnp.float32),
            pltpu.VMEM((_B, 128), jnp.float32),
            pltpu.VMEM((_B, 128), jnp.float32),
            pltpu.VMEM((_B, 128), jnp.int32),
        ],
        compiler_params=pltpu.CompilerParams(
            dimension_semantics=("arbitrary",)),
    )(hidden_states, w_lm)


_N_IDX = _B * _K


def _sc_remap_body(d2t_hbm, idx_hbm, out_hbm, idx_v, val_v, out_v, sem):
    wid = lax.axis_index("s") * 2 + lax.axis_index("c")
    base = wid * 16
    pltpu.sync_copy(idx_hbm.at[pl.ds(base, 16)], idx_v)
    pltpu.async_copy(d2t_hbm.at[idx_v], val_v, sem).wait()
    out_v[...] = idx_v[...] + val_v[...]
    pltpu.sync_copy(out_v, out_hbm.at[pl.ds(base, 16)])


@functools.cache
def _sc_remap():
    mesh = plsc.VectorSubcoreMesh(core_axis_name="c", subcore_axis_name="s")
    return pl.kernel(
        _sc_remap_body,
        mesh=mesh,
        out_type=jax.ShapeDtypeStruct((_N_IDX,), jnp.int32),
        scratch_types=[
            pltpu.VMEM((16,), jnp.int32),
            pltpu.VMEM((16,), jnp.int32),
            pltpu.VMEM((16,), jnp.int32),
            pltpu.SemaphoreType.DMA,
        ],
    )


def kernel(hidden_states, d2t, W_lm):
    topi, scores = _tc_topk(hidden_states, W_lm)
    mapped = _sc_remap()(d2t, topi.reshape(-1)).reshape(_B, _K)
    return mapped, scores
